# baseline (speedup 1.0000x reference)
import functools

import jax
import jax.numpy as jnp
from jax import lax
from jax.experimental import pallas as pl
from jax.experimental.pallas import tpu as pltpu
from jax.experimental.pallas import tpu_sc as plsc

N_NODES = 10000
N_EDGES = 160000
D_FEAT = 256

NC = 2
NS = 16
DH = D_FEAT // NC
EPT = N_EDGES // NS
EB = 80
NB = EPT // EB
NPAD = 10112
RPT = NPAD // NS
LASTR = N_NODES - (NS - 1) * RPT
NBUF = 4
KIDX = 8


def _mp_sc(x1, eidx):
    mesh = plsc.VectorSubcoreMesh(core_axis_name="c", subcore_axis_name="s")

    @functools.partial(
        pl.kernel,
        mesh=mesh,
        out_type=jax.ShapeDtypeStruct((N_NODES, D_FEAT), jnp.float32),
        scratch_types=(
            [pltpu.VMEM((EB,), jnp.int32) for _ in range(KIDX)]
            + [pltpu.VMEM((EB,), jnp.int32) for _ in range(KIDX)]
            + [pltpu.VMEM((EB, DH), jnp.float32) for _ in range(NBUF)]
            + [pltpu.VMEM((40, DH), jnp.float32)]
            + [pltpu.VMEM_SHARED((NPAD, DH), jnp.float32)]
            + [pltpu.SemaphoreType.DMA for _ in range(KIDX)]
            + [pltpu.SemaphoreType.DMA]
            + [pltpu.SemaphoreType.DMA for _ in range(2 * NBUF)]
        ),
    )
    def body(x_hbm, e_hbm, out_hbm, *refs):
        sring = refs[0:KIDX]
        dring = refs[KIDX:2 * KIDX]
        rows = refs[2 * KIDX:2 * KIDX + NBUF]
        zbuf = refs[2 * KIDX + NBUF]
        acc = refs[2 * KIDX + NBUF + 1]
        isem = refs[2 * KIDX + NBUF + 2:3 * KIDX + NBUF + 2]
        zsem = refs[3 * KIDX + NBUF + 2]
        gsem = refs[3 * KIDX + NBUF + 3:3 * KIDX + 2 * NBUF + 3]
        ssem = refs[3 * KIDX + 2 * NBUF + 3:3 * KIDX + 3 * NBUF + 3]

        c = lax.axis_index("c")
        s = lax.axis_index("s")

        def sslice(b):
            off = pl.multiple_of(s * EPT + b * EB, 8)
            return e_hbm.at[pl.ds(off, EB)]

        def dslice(b):
            off = pl.multiple_of(N_EDGES + s * EPT + b * EB, 8)
            return e_hbm.at[pl.ds(off, EB)]

        def idx_start(b, k):
            pltpu.async_copy(sslice(b), sring[k], isem[k])
            pltpu.async_copy(dslice(b), dring[k], isem[k])

        def idx_wait(b, k):
            pltpu.make_async_copy(sslice(b), sring[k], isem[k]).wait()
            pltpu.make_async_copy(dslice(b), dring[k], isem[k]).wait()
            for t in range(EB // 16):
                v = sring[k][pl.ds(t * 16, 16)]
                sring[k][pl.ds(t * 16, 16)] = v * 2 + c

        def gather_start(j, k):
            pltpu.async_copy(x_hbm.at[sring[k]], rows[j], gsem[j])

        def gather_wait(j):
            pltpu.make_async_copy(x_hbm.at[sring[0]], rows[j],
                                  gsem[j]).wait()

        def scatter_start(j, k):
            pltpu.async_copy(rows[j], acc.at[dring[k]], ssem[j], add=True)

        def scatter_wait(j):
            pltpu.make_async_copy(rows[0], acc.at[dring[0]],
                                  ssem[j]).wait()

        for b in range(4):
            idx_start(b, b)

        def zstore(t, carry):
            zbuf[t // 8, pl.ds((t % 8) * 16, 16)] = jnp.zeros((16,),
                                                              jnp.float32)
            return carry

        lax.fori_loop(0, 40 * (DH // 16), zstore, 0)

        def zcopy(k, n):
            off = pl.multiple_of(s * RPT + k * 40, 8)
            return zbuf.at[pl.ds(0, n)], acc.at[pl.ds(off, n)]

        for k in range(15):
            pltpu.async_copy(*zcopy(k, 40), zsem)
        pltpu.async_copy(*zcopy(15, 32), zsem)

        idx_wait(0, 0)
        idx_wait(1, 1)
        gather_start(0, 0)
        gather_start(1, 1)
        for k in range(15):
            pltpu.make_async_copy(*zcopy(k, 40), zsem).wait()
        pltpu.make_async_copy(*zcopy(15, 32), zsem).wait()
        plsc.subcore_barrier()

        def slot(b, head=False, idx=True, gather=True):
            if gather:
                idx_wait(b + 2, (b + 2) % KIDX)
            if not head:
                scatter_wait((b - 2) % NBUF)
            if gather:
                gather_start((b + 2) % NBUF, (b + 2) % KIDX)
            gather_wait(b % NBUF)
            scatter_start(b % NBUF, b % KIDX)
            if idx:
                idx_start(b + 4, (b + 4) % KIDX)

        slot(0, head=True)
        slot(1, head=True)

        def step(ii, carry):
            base = 2 + ii * KIDX
            for j in range(KIDX):
                bb = 2 + j
                idx_wait(base + j + 2, (bb + 2) % KIDX)
                scatter_wait((bb - 2) % NBUF)
                gather_start((bb + 2) % NBUF, (bb + 2) % KIDX)
                gather_wait(bb % NBUF)
                scatter_start(bb % NBUF, bb % KIDX)
                idx_start(base + j + 4, (bb + 4) % KIDX)
            return carry

        lax.fori_loop(0, 14, step, 0)
        for b in range(114, NB - 4):
            slot(b)
        slot(NB - 4, idx=False)
        slot(NB - 3, idx=False)
        slot(NB - 2, idx=False, gather=False)
        slot(NB - 1, idx=False, gather=False)
        scatter_wait((NB - 2) % NBUF)
        scatter_wait((NB - 1) % NBUF)
        plsc.subcore_barrier()

        coff = pl.multiple_of(c * DH, 128)
        aoff = pl.multiple_of(s * RPT, 8)

        @pl.when(s < NS - 1)
        def _():
            pltpu.sync_copy(acc.at[pl.ds(aoff, RPT)],
                            out_hbm.at[pl.ds(aoff, RPT), pl.ds(coff, DH)])

        @pl.when(s == NS - 1)
        def _():
            pltpu.sync_copy(acc.at[pl.ds(aoff, LASTR)],
                            out_hbm.at[pl.ds(aoff, LASTR), pl.ds(coff, DH)])

    return body(x1, eidx)


def kernel(x, edge_index):
    x1 = x.reshape(2 * N_NODES, DH)
    eidx = edge_index.astype(jnp.int32).reshape(2 * N_EDGES)
    return _mp_sc(x1, eidx)


# speedup vs baseline: 1.0053x; 1.0053x over previous
import functools

import jax
import jax.numpy as jnp
from jax import lax
from jax.experimental import pallas as pl
from jax.experimental.pallas import tpu as pltpu
from jax.experimental.pallas import tpu_sc as plsc

N_NODES = 10000
N_EDGES = 160000
D_FEAT = 256

NC = 2
NS = 16
DH = D_FEAT // NC
EPT = N_EDGES // NS
EB = 80
NB = EPT // EB
NPAD = 10112
RPT = NPAD // NS
LASTR = N_NODES - (NS - 1) * RPT
NBUF = 4
KIDX = 8


def _mp_sc(x1, eidx):
    mesh = plsc.VectorSubcoreMesh(core_axis_name="c", subcore_axis_name="s")

    @functools.partial(
        pl.kernel,
        mesh=mesh,
        out_type=jax.ShapeDtypeStruct((N_NODES, D_FEAT), jnp.float32),
        scratch_types=(
            [pltpu.VMEM((EB,), jnp.int32) for _ in range(KIDX)]
            + [pltpu.VMEM((EB,), jnp.int32) for _ in range(KIDX)]
            + [pltpu.VMEM((EB, DH), jnp.float32) for _ in range(NBUF)]
            + [pltpu.VMEM((40, DH), jnp.float32)]
            + [pltpu.VMEM_SHARED((NPAD, DH), jnp.float32)]
            + [pltpu.SemaphoreType.DMA for _ in range(KIDX)]
            + [pltpu.SemaphoreType.DMA]
            + [pltpu.SemaphoreType.DMA for _ in range(2 * NBUF)]
        ),
    )
    def body(x_hbm, e_hbm, out_hbm, *refs):
        sring = refs[0:KIDX]
        dring = refs[KIDX:2 * KIDX]
        rows = refs[2 * KIDX:2 * KIDX + NBUF]
        zbuf = refs[2 * KIDX + NBUF]
        acc = refs[2 * KIDX + NBUF + 1]
        isem = refs[2 * KIDX + NBUF + 2:3 * KIDX + NBUF + 2]
        zsem = refs[3 * KIDX + NBUF + 2]
        gsem = refs[3 * KIDX + NBUF + 3:3 * KIDX + 2 * NBUF + 3]
        ssem = refs[3 * KIDX + 2 * NBUF + 3:3 * KIDX + 3 * NBUF + 3]

        c = lax.axis_index("c")
        s = lax.axis_index("s")

        def sslice(b):
            off = pl.multiple_of(s * EPT + b * EB, 8)
            return e_hbm.at[pl.ds(off, EB)]

        def dslice(b):
            off = pl.multiple_of(N_EDGES + s * EPT + b * EB, 8)
            return e_hbm.at[pl.ds(off, EB)]

        def idx_start(b, k):
            pltpu.async_copy(sslice(b), sring[k], isem[k])
            pltpu.async_copy(dslice(b), dring[k], isem[k])

        def idx_wait(b, k):
            pltpu.make_async_copy(sslice(b), sring[k], isem[k]).wait()
            pltpu.make_async_copy(dslice(b), dring[k], isem[k]).wait()
            for t in range(EB // 16):
                v = sring[k][pl.ds(t * 16, 16)]
                sring[k][pl.ds(t * 16, 16)] = v * 2 + c

        def gather_start(j, k):
            pltpu.async_copy(x_hbm.at[sring[k]], rows[j], gsem[j])

        def gather_wait(j):
            pltpu.make_async_copy(x_hbm.at[sring[0]], rows[j],
                                  gsem[j]).wait()

        def scatter_start(j, k):
            pltpu.async_copy(rows[j], acc.at[dring[k]], ssem[j], add=True)

        def scatter_wait(j):
            pltpu.make_async_copy(rows[0], acc.at[dring[0]],
                                  ssem[j]).wait()

        for b in range(4):
            idx_start(b, b)

        def zstore(t, carry):
            zbuf[t // 8, pl.ds((t % 8) * 16, 16)] = jnp.zeros((16,),
                                                              jnp.float32)
            return carry

        lax.fori_loop(0, 40 * (DH // 16), zstore, 0)

        def zcopy(k, n):
            off = pl.multiple_of(s * RPT + k * 40, 8)
            return zbuf.at[pl.ds(0, n)], acc.at[pl.ds(off, n)]

        for k in range(15):
            pltpu.async_copy(*zcopy(k, 40), zsem)
        pltpu.async_copy(*zcopy(15, 32), zsem)

        idx_wait(0, 0)
        idx_wait(1, 1)
        gather_start(0, 0)
        gather_start(1, 1)
        for k in range(15):
            pltpu.make_async_copy(*zcopy(k, 40), zsem).wait()
        pltpu.make_async_copy(*zcopy(15, 32), zsem).wait()
        plsc.subcore_barrier()

        def slot(b, head=False, idx=True, gather=True):
            if gather:
                idx_wait(b + 2, (b + 2) % KIDX)
            if not head:
                scatter_wait((b - 2) % NBUF)
            if gather:
                gather_start((b + 2) % NBUF, (b + 2) % KIDX)
            gather_wait(b % NBUF)
            scatter_start(b % NBUF, b % KIDX)
            if idx:
                idx_start(b + 4, (b + 4) % KIDX)

        slot(0, head=True)
        slot(1, head=True)

        def step(ii, carry):
            base = 2 + ii * KIDX
            for j in range(KIDX):
                bb = 2 + j
                idx_wait(base + j + 2, (bb + 2) % KIDX)
                scatter_wait((bb - 2) % NBUF)
                gather_start((bb + 2) % NBUF, (bb + 2) % KIDX)
                gather_wait(bb % NBUF)
                scatter_start(bb % NBUF, bb % KIDX)

                @pl.when(base + j + 4 < NB)
                def _():
                    idx_start(base + j + 4, (bb + 4) % KIDX)
            return carry

        lax.fori_loop(0, (NB - 5) // KIDX, step, 0)

        slot(NB - 3, idx=False)
        slot(NB - 2, idx=False, gather=False)
        slot(NB - 1, idx=False, gather=False)
        scatter_wait((NB - 2) % NBUF)
        scatter_wait((NB - 1) % NBUF)
        plsc.subcore_barrier()

        coff = pl.multiple_of(c * DH, 128)
        aoff = pl.multiple_of(s * RPT, 8)

        @pl.when(s < NS - 1)
        def _():
            pltpu.sync_copy(acc.at[pl.ds(aoff, RPT)],
                            out_hbm.at[pl.ds(aoff, RPT), pl.ds(coff, DH)])

        @pl.when(s == NS - 1)
        def _():
            pltpu.sync_copy(acc.at[pl.ds(aoff, LASTR)],
                            out_hbm.at[pl.ds(aoff, LASTR), pl.ds(coff, DH)])

    return body(x1, eidx)


def kernel(x, edge_index):
    x1 = x.reshape(2 * N_NODES, DH)
    eidx = edge_index.astype(jnp.int32).reshape(2 * N_EDGES)
    return _mp_sc(x1, eidx)
